# Initial kernel scaffold; baseline (speedup 1.0000x reference)
import functools

import jax
import jax.numpy as jnp
from jax import lax
from jax.experimental import pallas as pl
from jax.experimental.pallas import tpu as pltpu
from jax.experimental.pallas import tpu_sc as plsc

N = 6_400_000
S_SEG = 10_000
L = 16
NW = 32
SPAD = 10_240
COLS = SPAD // NW
CHUNK = N // NW
BLK = 20_000
NBLK = CHUNK // BLK
NVEC = BLK // L

_MESH = dict(core_axis_name="c", subcore_axis_name="s")


def _take(x, idx):
    return jnp.take_along_axis(x, idx, axis=0)


def _phase1(data, ids):
    mesh = plsc.VectorSubcoreMesh(**_MESH)

    @functools.partial(
        pl.kernel,
        out_type=jax.ShapeDtypeStruct((NW, SPAD), jnp.float32),
        mesh=mesh,
        scratch_types=[
            pltpu.VMEM((2, BLK), jnp.float32),
            pltpu.VMEM((2, BLK), jnp.int32),
            pltpu.VMEM((SPAD,), jnp.float32),
            pltpu.SemaphoreType.DMA,
            pltpu.SemaphoreType.DMA,
            pltpu.SemaphoreType.DMA,
            pltpu.SemaphoreType.DMA,
            pltpu.SemaphoreType.DMA,
        ],
    )
    def k(data_hbm, ids_hbm, part_hbm, dbuf, ibuf, acc, sd0, sd1, si0, si1,
          so):
        wid = lax.axis_index("c") * 16 + lax.axis_index("s")
        base = wid * CHUNK
        dsem = (sd0, sd1)
        isem = (si0, si1)

        minf = jnp.full((L,), -jnp.inf, dtype=jnp.float32)

        def ibody(i, c):
            acc[pl.ds(i * L, L)] = minf
            return c

        lax.fori_loop(0, SPAD // L, ibody, 0)

        def issue(b):
            slot = b % 2
            off = base + b * BLK
            pltpu.async_copy(data_hbm.at[pl.ds(off, BLK)], dbuf.at[slot],
                             dsem[slot])
            pltpu.async_copy(ids_hbm.at[pl.ds(off, BLK)], ibuf.at[slot],
                             isem[slot])

        def wait(b):
            slot = b % 2
            off = base + b * BLK
            pltpu.make_async_copy(data_hbm.at[pl.ds(off, BLK)], dbuf.at[slot],
                                  dsem[slot]).wait()
            pltpu.make_async_copy(ids_hbm.at[pl.ds(off, BLK)], ibuf.at[slot],
                                  isem[slot]).wait()

        iota = lax.iota(jnp.int32, L)
        last = jnp.full((L,), L - 1, dtype=jnp.int32)

        def process(slot):
            def vbody(v, c):
                off = v * L
                seg = ibuf[slot, pl.ds(off, L)]
                vals = dbuf[slot, pl.ds(off, L)]
                for sh in (1, 2, 4, 8):
                    pidx = jnp.maximum(iota - sh, 0)
                    gseg = _take(seg, pidx)
                    gval = _take(vals, pidx)
                    vals = jnp.where(seg == gseg, jnp.maximum(vals, gval),
                                     vals)
                nseg = _take(seg, jnp.minimum(iota + 1, last))
                end = (seg != nseg) | (iota == last)
                cur = plsc.load_gather(acc, [seg])
                plsc.store_scatter(acc, [seg], jnp.maximum(cur, vals),
                                   mask=end)
                return c

            lax.fori_loop(0, NVEC, vbody, 0)

        issue(0)
        for b in range(NBLK):
            if b + 1 < NBLK:
                issue(b + 1)
            wait(b)
            process(b % 2)

        pltpu.async_copy(acc, part_hbm.at[wid], so).wait()

    return k(data, ids)


def _phase2(part):
    mesh = plsc.VectorSubcoreMesh(**_MESH)

    @functools.partial(
        pl.kernel,
        out_type=jax.ShapeDtypeStruct((SPAD,), jnp.float32),
        mesh=mesh,
        scratch_types=[
            pltpu.VMEM((NW, COLS), jnp.float32),
            pltpu.VMEM((COLS,), jnp.float32),
            pltpu.SemaphoreType.DMA,
        ],
    )
    def k(part_hbm, out_hbm, buf, obuf, sem):
        wid = lax.axis_index("c") * 16 + lax.axis_index("s")
        col0 = wid * COLS
        for r in range(NW):
            pltpu.async_copy(part_hbm.at[r, pl.ds(col0, COLS)], buf.at[r],
                             sem)
        for r in range(NW):
            pltpu.make_async_copy(part_hbm.at[r, pl.ds(col0, COLS)],
                                  buf.at[r], sem).wait()

        def cbody(j, c):
            off = j * L
            m = buf[0, pl.ds(off, L)]
            for r in range(1, NW):
                m = jnp.maximum(m, buf[r, pl.ds(off, L)])
            obuf[pl.ds(off, L)] = m
            return c

        lax.fori_loop(0, COLS // L, cbody, 0)
        pltpu.async_copy(obuf, out_hbm.at[pl.ds(col0, COLS)], sem).wait()

    return k(part)


def kernel(data, segment_ids, num_segments):
    del num_segments
    ids = segment_ids.astype(jnp.int32)
    part = _phase1(data, ids)
    out = _phase2(part)
    return out[:S_SEG]


# speedup vs baseline: 4.4850x; 4.4850x over previous
import functools

import jax
import jax.numpy as jnp
from jax import lax
from jax.experimental import pallas as pl
from jax.experimental.pallas import tpu as pltpu
from jax.experimental.pallas import tpu_sc as plsc

N = 6_400_000
S_SEG = 10_000
L = 16
NW = 32
SPAD = 10_240
COLS = SPAD // NW
CHUNK = N // NW
BLK = 20_000
NBLK = CHUNK // BLK
NVEC = BLK // L

_MESH = dict(core_axis_name="c", subcore_axis_name="s")
_PARAMS = pltpu.CompilerParams(
    needs_layout_passes=False, use_tc_tiling_on_sc=False
)


def _take(x, idx):
    return jnp.take_along_axis(x, idx, axis=0)


def _phase1(data, ids):
    mesh = plsc.VectorSubcoreMesh(**_MESH)

    @functools.partial(
        pl.kernel,
        out_type=jax.ShapeDtypeStruct((NW, SPAD), jnp.float32),
        mesh=mesh,
        scratch_types=[
            pltpu.VMEM((2, BLK), jnp.float32),
            pltpu.VMEM((2, BLK), jnp.int32),
            pltpu.VMEM((SPAD,), jnp.float32),
            pltpu.SemaphoreType.DMA,
            pltpu.SemaphoreType.DMA,
            pltpu.SemaphoreType.DMA,
            pltpu.SemaphoreType.DMA,
            pltpu.SemaphoreType.DMA,
        ],
        compiler_params=_PARAMS,
    )
    def k(data_hbm, ids_hbm, part_hbm, dbuf, ibuf, acc, sd0, sd1, si0, si1,
          so):
        wid = lax.axis_index("c") * 16 + lax.axis_index("s")
        base = wid * CHUNK
        dsem = (sd0, sd1)
        isem = (si0, si1)

        minf = jnp.full((L,), -jnp.inf, dtype=jnp.float32)

        def ibody(i, c):
            acc[pl.ds(i * L, L)] = minf
            return c

        lax.fori_loop(0, SPAD // L, ibody, 0)

        def issue(b):
            slot = b % 2
            off = base + b * BLK
            pltpu.async_copy(data_hbm.at[pl.ds(off, BLK)], dbuf.at[slot],
                             dsem[slot])
            pltpu.async_copy(ids_hbm.at[pl.ds(off, BLK)], ibuf.at[slot],
                             isem[slot])

        def wait(b):
            slot = b % 2
            off = base + b * BLK
            pltpu.make_async_copy(data_hbm.at[pl.ds(off, BLK)], dbuf.at[slot],
                                  dsem[slot]).wait()
            pltpu.make_async_copy(ids_hbm.at[pl.ds(off, BLK)], ibuf.at[slot],
                                  isem[slot]).wait()

        iota = lax.iota(jnp.int32, L)
        last = jnp.full((L,), L - 1, dtype=jnp.int32)

        def process(slot):
            def vbody(v, c):
                off = v * L
                seg = ibuf[slot, pl.ds(off, L)]
                vals = dbuf[slot, pl.ds(off, L)]
                for sh in (1, 2, 4, 8):
                    pidx = jnp.maximum(iota - sh, 0)
                    gseg = _take(seg, pidx)
                    gval = _take(vals, pidx)
                    vals = jnp.where(seg == gseg, jnp.maximum(vals, gval),
                                     vals)
                nseg = _take(seg, jnp.minimum(iota + 1, last))
                end = (seg != nseg) | (iota == last)
                cur = plsc.load_gather(acc, [seg])
                plsc.store_scatter(acc, [seg], jnp.maximum(cur, vals),
                                   mask=end)
                return c

            lax.fori_loop(0, NVEC, vbody, 0)

        issue(0)
        for b in range(NBLK):
            if b + 1 < NBLK:
                issue(b + 1)
            wait(b)
            process(b % 2)

        pltpu.async_copy(acc, part_hbm.at[wid], so).wait()

    return k(data, ids)


def _phase2(part):
    mesh = plsc.VectorSubcoreMesh(**_MESH)

    @functools.partial(
        pl.kernel,
        out_type=jax.ShapeDtypeStruct((SPAD,), jnp.float32),
        mesh=mesh,
        scratch_types=[
            pltpu.VMEM((NW, COLS), jnp.float32),
            pltpu.VMEM((COLS,), jnp.float32),
            pltpu.SemaphoreType.DMA,
        ],
        compiler_params=_PARAMS,
    )
    def k(part_hbm, out_hbm, buf, obuf, sem):
        wid = lax.axis_index("c") * 16 + lax.axis_index("s")
        col0 = wid * COLS
        for r in range(NW):
            pltpu.async_copy(part_hbm.at[r, pl.ds(col0, COLS)], buf.at[r],
                             sem)
        for r in range(NW):
            pltpu.make_async_copy(part_hbm.at[r, pl.ds(col0, COLS)],
                                  buf.at[r], sem).wait()

        def cbody(j, c):
            off = j * L
            m = buf[0, pl.ds(off, L)]
            for r in range(1, NW):
                m = jnp.maximum(m, buf[r, pl.ds(off, L)])
            obuf[pl.ds(off, L)] = m
            return c

        lax.fori_loop(0, COLS // L, cbody, 0)
        pltpu.async_copy(obuf, out_hbm.at[pl.ds(col0, COLS)], sem).wait()

    return k(part)


def kernel(data, segment_ids, num_segments):
    del num_segments
    ids = segment_ids.astype(jnp.int32)
    part = _phase1(data, ids)
    out = _phase2(part)
    return out[:S_SEG]


# speedup vs baseline: 6.5932x; 1.4701x over previous
import functools

import jax
import jax.numpy as jnp
from jax import lax
from jax.experimental import pallas as pl
from jax.experimental.pallas import tpu as pltpu
from jax.experimental.pallas import tpu_sc as plsc

N = 6_400_000
S_SEG = 10_000
L = 16
NW = 32
SPAD = 10_240
COLS = SPAD // NW
CHUNK = N // NW
BLK = 8_000
NBLK = CHUNK // BLK
GRP = 4 * L
NGRP = BLK // GRP

_MESH = dict(core_axis_name="c", subcore_axis_name="s")
_PARAMS = pltpu.CompilerParams(
    needs_layout_passes=False, use_tc_tiling_on_sc=False
)


def _take(x, idx):
    return jnp.take_along_axis(x, idx, axis=0)


def _phase1(data, ids):
    mesh = plsc.VectorSubcoreMesh(**_MESH)

    @functools.partial(
        pl.kernel,
        out_type=jax.ShapeDtypeStruct((NW, SPAD), jnp.float32),
        mesh=mesh,
        scratch_types=[
            pltpu.VMEM((2, BLK), jnp.float32),
            pltpu.VMEM((2, BLK), jnp.int32),
            pltpu.VMEM((SPAD,), jnp.float32),
            pltpu.SemaphoreType.DMA,
            pltpu.SemaphoreType.DMA,
            pltpu.SemaphoreType.DMA,
            pltpu.SemaphoreType.DMA,
            pltpu.SemaphoreType.DMA,
        ],
        compiler_params=_PARAMS,
    )
    def k(data_hbm, ids_hbm, part_hbm, dbuf, ibuf, acc, sd0, sd1, si0, si1,
          so):
        wid = lax.axis_index("c") * 16 + lax.axis_index("s")
        base = wid * CHUNK
        dsem = (sd0, sd1)
        isem = (si0, si1)

        minf = jnp.full((L,), -jnp.inf, dtype=jnp.float32)
        iota = lax.iota(jnp.int32, L)
        last = jnp.full((L,), L - 1, dtype=jnp.int32)
        lane0 = iota == 0

        def ibody(i, c):
            acc[pl.ds(i * L, L)] = minf
            return c

        lax.fori_loop(0, SPAD // L, ibody, 0)

        def issue(b):
            slot = b % 2
            off = base + b * BLK
            pltpu.async_copy(data_hbm.at[pl.ds(off, BLK)], dbuf.at[slot],
                             dsem[slot])
            pltpu.async_copy(ids_hbm.at[pl.ds(off, BLK)], ibuf.at[slot],
                             isem[slot])

        def wait(b):
            slot = b % 2
            off = base + b * BLK
            pltpu.make_async_copy(data_hbm.at[pl.ds(off, BLK)], dbuf.at[slot],
                                  dsem[slot]).wait()
            pltpu.make_async_copy(ids_hbm.at[pl.ds(off, BLK)], ibuf.at[slot],
                                  isem[slot]).wait()

        def flush(carry_id, carry_vec):
            cidv = jnp.full((L,), carry_id, dtype=jnp.int32)
            red = jnp.full((L,), jnp.max(carry_vec), dtype=jnp.float32)
            cur = plsc.load_gather(acc, [cidv])
            plsc.store_scatter(acc, [cidv], jnp.maximum(cur, red), mask=lane0)

        def general_vec(slot, off):
            seg = ibuf[slot, pl.ds(off, L)]
            vals = dbuf[slot, pl.ds(off, L)]
            for sh in (1, 2, 4, 8):
                pidx = jnp.maximum(iota - sh, 0)
                gseg = _take(seg, pidx)
                gval = _take(vals, pidx)
                vals = jnp.where(seg == gseg, jnp.maximum(vals, gval), vals)
            nseg = _take(seg, jnp.minimum(iota + 1, last))
            end = (seg != nseg) | (iota == last)
            cur = plsc.load_gather(acc, [seg])
            plsc.store_scatter(acc, [seg], jnp.maximum(cur, vals), mask=end)

        def process(slot, carry):
            def gbody(g, carry):
                carry_id, carry_vec = carry
                off = g * GRP
                first = ibuf[slot, pl.ds(off, L)][0]
                lastid = ibuf[slot, pl.ds(off + GRP - L, L)][L - 1]
                is_fast = (first == carry_id) & (lastid == carry_id)

                def fast(carry_vec):
                    x0 = dbuf[slot, pl.ds(off, L)]
                    x1 = dbuf[slot, pl.ds(off + L, L)]
                    x2 = dbuf[slot, pl.ds(off + 2 * L, L)]
                    x3 = dbuf[slot, pl.ds(off + 3 * L, L)]
                    m = jnp.maximum(jnp.maximum(x0, x1),
                                    jnp.maximum(x2, x3))
                    return carry_id, jnp.maximum(carry_vec, m)

                def slow(carry_vec):
                    flush(carry_id, carry_vec)
                    for v in range(GRP // L):
                        general_vec(slot, off + v * L)
                    return lastid, minf

                return lax.cond(is_fast, fast, slow, carry_vec)

            return lax.fori_loop(0, NGRP, gbody, carry)

        issue(0)
        wait(0)
        carry = (ibuf[0, pl.ds(0, L)][0], minf)
        for b in range(NBLK):
            if b + 1 < NBLK:
                issue(b + 1)
            if b > 0:
                wait(b)
            carry = process(b % 2, carry)
        flush(*carry)

        pltpu.async_copy(acc, part_hbm.at[wid], so).wait()

    return k(data, ids)


def _phase2(part):
    mesh = plsc.VectorSubcoreMesh(**_MESH)

    @functools.partial(
        pl.kernel,
        out_type=jax.ShapeDtypeStruct((SPAD,), jnp.float32),
        mesh=mesh,
        scratch_types=[
            pltpu.VMEM((NW, COLS), jnp.float32),
            pltpu.VMEM((COLS,), jnp.float32),
            pltpu.SemaphoreType.DMA,
        ],
        compiler_params=_PARAMS,
    )
    def k(part_hbm, out_hbm, buf, obuf, sem):
        wid = lax.axis_index("c") * 16 + lax.axis_index("s")
        col0 = wid * COLS
        for r in range(NW):
            pltpu.async_copy(part_hbm.at[r, pl.ds(col0, COLS)], buf.at[r],
                             sem)
        for r in range(NW):
            pltpu.make_async_copy(part_hbm.at[r, pl.ds(col0, COLS)],
                                  buf.at[r], sem).wait()

        def cbody(j, c):
            off = j * L
            m = buf[0, pl.ds(off, L)]
            for r in range(1, NW):
                m = jnp.maximum(m, buf[r, pl.ds(off, L)])
            obuf[pl.ds(off, L)] = m
            return c

        lax.fori_loop(0, COLS // L, cbody, 0)
        pltpu.async_copy(obuf, out_hbm.at[pl.ds(col0, COLS)], sem).wait()

    return k(part)


def kernel(data, segment_ids, num_segments):
    del num_segments
    ids = segment_ids.astype(jnp.int32)
    part = _phase1(data, ids)
    out = _phase2(part)
    return out[:S_SEG]
